# Initial kernel scaffold; baseline (speedup 1.0000x reference)
import jax
import jax.numpy as jnp
from jax import lax
from jax.experimental import pallas as pl
from jax.experimental.pallas import tpu as pltpu
from jax.experimental.pallas import tpu_sc as plsc

N = 10000
D = 128
E = 320000

NC = 2
NS = 16
NW = NC * NS
L = 128
CH = 80
E_PAD = NW * CH * L
ACC_N = 10016
STRIPE = ACC_N // NS
CW = 16

_mesh = plsc.VectorSubcoreMesh(core_axis_name="c", subcore_axis_name="s")


def _make_segsum(with_cnt):
  out_type = [jax.ShapeDtypeStruct((NC, ACC_N, D), jnp.float32)]
  scratch = [
      pltpu.VMEM((CH, L), jnp.int32),
      pltpu.VMEM((CH, L), jnp.int32),
      pltpu.VMEM((L, D), jnp.float32),
      pltpu.VMEM((L, D), jnp.float32),
      pltpu.VMEM_SHARED((ACC_N, D), jnp.float32),
      pltpu.SemaphoreType.DMA,
      pltpu.SemaphoreType.DMA,
  ]
  if with_cnt:
    out_type.append(jax.ShapeDtypeStruct((NC, ACC_N, CW), jnp.float32))
    scratch += [
        pltpu.VMEM((L, CW), jnp.float32),
        pltpu.VMEM_SHARED((ACC_N, CW), jnp.float32),
    ]

  def body(x_hbm, src_hbm, dst_hbm, z_hbm, zc_hbm, ones_hbm,
           out_sum, out_cnt, src_v, dst_v, buf0, buf1, acc, sg0, sg1,
           ones_v=None, cacc=None):
    cid = lax.axis_index("c")
    sid = lax.axis_index("s")
    tid = cid * NS + sid

    pltpu.sync_copy(src_hbm.at[tid], src_v)
    pltpu.sync_copy(dst_hbm.at[tid], dst_v)
    pltpu.sync_copy(z_hbm.at[pl.ds(sid * STRIPE, STRIPE)],
                    acc.at[pl.ds(sid * STRIPE, STRIPE)])
    if with_cnt:
      pltpu.sync_copy(zc_hbm.at[pl.ds(sid * STRIPE, STRIPE)],
                      cacc.at[pl.ds(sid * STRIPE, STRIPE)])
      pltpu.sync_copy(ones_hbm, ones_v)
    plsc.subcore_barrier()

    def gather(j, buf, sem):
      pltpu.async_copy(x_hbm.at[src_v.at[j]], buf, sem)

    def gather_wait(j, buf, sem):
      pltpu.make_async_copy(x_hbm.at[src_v.at[j]], buf, sem).wait()

    def scatter(j, buf):
      pltpu.sync_copy(buf, acc.at[dst_v.at[j]], add=True)
      if with_cnt:
        pltpu.sync_copy(ones_v, cacc.at[dst_v.at[j]], add=True)

    gather(0, buf0, sg0)
    nsteps = CH // 2

    def step(t, carry):
      j0 = 2 * t
      j1 = j0 + 1
      gather(j1, buf1, sg1)
      gather_wait(j0, buf0, sg0)
      scatter(j0, buf0)

      @pl.when(t + 1 < nsteps)
      def _():
        gather(j0 + 2, buf0, sg0)
      gather_wait(j1, buf1, sg1)
      scatter(j1, buf1)
      return carry

    lax.fori_loop(0, nsteps, step, 0)
    plsc.subcore_barrier()

    sl = pl.ds(sid * STRIPE, STRIPE)
    pltpu.sync_copy(acc.at[sl], out_sum.at[cid, sl])
    if with_cnt:
      pltpu.sync_copy(cacc.at[sl], out_cnt.at[cid, sl])

  if with_cnt:
    def body_cnt(x_hbm, src_hbm, dst_hbm, z_hbm, zc_hbm, ones_hbm,
                 out_sum, out_cnt, src_v, dst_v, buf0, buf1, acc, sg0, sg1,
                 ones_v, cacc):
      body(x_hbm, src_hbm, dst_hbm, z_hbm, zc_hbm, ones_hbm,
           out_sum, out_cnt, src_v, dst_v, buf0, buf1, acc, sg0, sg1,
           ones_v, cacc)
    fn = body_cnt
  else:
    def body_nocnt(x_hbm, src_hbm, dst_hbm, z_hbm, zc_hbm, ones_hbm,
                   out_sum, src_v, dst_v, buf0, buf1, acc, sg0, sg1):
      body(x_hbm, src_hbm, dst_hbm, z_hbm, zc_hbm, ones_hbm,
           out_sum, None, src_v, dst_v, buf0, buf1, acc, sg0, sg1)
    fn = body_nocnt

  return pl.kernel(fn, out_type=tuple(out_type) if with_cnt else out_type[0],
                   mesh=_mesh, scratch_types=scratch)


_segsum_cnt = _make_segsum(True)
_segsum = _make_segsum(False)

R = 400


def _combine_body(p_ref, c_ref, x_ref, w_ref, b_ref, o_ref):
  s = p_ref[0] + p_ref[1]
  cnt = c_ref[0, :, 0:1] + c_ref[1, :, 0:1]
  agg = s / jnp.maximum(cnt, 1.0)
  cat = jnp.concatenate([agg, x_ref[...]], axis=1)
  o = lax.dot_general(cat, w_ref[...], (((1,), (0,)), ((), ())),
                      preferred_element_type=jnp.float32,
                      precision=lax.Precision.HIGHEST)
  o_ref[...] = jnp.maximum(o + b_ref[...], 0.0)


_combine = pl.pallas_call(
    _combine_body,
    grid=(N // R,),
    in_specs=[
        pl.BlockSpec((NC, R, D), lambda i: (0, i, 0)),
        pl.BlockSpec((NC, R, CW), lambda i: (0, i, 0)),
        pl.BlockSpec((R, D), lambda i: (i, 0)),
        pl.BlockSpec((2 * D, D), lambda i: (0, 0)),
        pl.BlockSpec((1, D), lambda i: (0, 0)),
    ],
    out_specs=pl.BlockSpec((R, D), lambda i: (i, 0)),
    out_shape=jax.ShapeDtypeStruct((N, D), jnp.float32),
)


def kernel(x, edge_index, edge_attr, W1l, W1r, b1, W2l, W2r, b2):
  npad = E_PAD - E
  src = jnp.concatenate(
      [edge_index[0].astype(jnp.int32),
       jnp.arange(npad, dtype=jnp.int32) % N]).reshape(NW, CH, L)
  dst = jnp.concatenate(
      [edge_index[1].astype(jnp.int32),
       N + (jnp.arange(npad, dtype=jnp.int32) % (ACC_N - N))]).reshape(NW, CH, L)
  z = jnp.zeros((ACC_N, D), jnp.float32)
  zc = jnp.zeros((ACC_N, CW), jnp.float32)
  ones = jnp.ones((L, CW), jnp.float32)
  w1 = jnp.concatenate([W1l.T, W1r.T], axis=0)
  w2 = jnp.concatenate([W2l.T, W2r.T], axis=0)

  p1, cnt = _segsum_cnt(x, src, dst, z, zc, ones)
  h1 = _combine(p1, cnt, x, w1, b1.reshape(1, D))
  p2 = _segsum(h1, src, dst, z, zc, ones)
  h2 = _combine(p2, cnt, h1, w2, b2.reshape(1, D))
  return h2


# speedup vs baseline: 6.4632x; 6.4632x over previous
import jax
import jax.numpy as jnp
from jax import lax
from jax.experimental import pallas as pl
from jax.experimental.pallas import tpu as pltpu
from jax.experimental.pallas import tpu_sc as plsc

N = 10000
D = 128
D2 = D // 2
E = 320000

NC = 2
NS = 16
NW = NC * NS
L = 128
E_PAD = 323584
CH = E_PAD // (NS * L)
CHC = E_PAD // (NW * L)
ACC_N = 10112
STRIPE = ACC_N // NS
CW = 16

_mesh = plsc.VectorSubcoreMesh(core_axis_name="c", subcore_axis_name="s")


def _segsum_body(xs_hbm, src_hbm, dst_hbm, z_hbm,
                 out_sum, src_v, dst_v, buf0, buf1, acc, sg0, sg1):
  cid = lax.axis_index("c")
  sid = lax.axis_index("s")

  pltpu.sync_copy(src_hbm.at[sid], src_v)
  pltpu.sync_copy(dst_hbm.at[sid], dst_v)
  sl = pl.ds(sid * STRIPE, STRIPE)
  pltpu.sync_copy(z_hbm.at[sl], acc.at[sl])
  plsc.subcore_barrier()

  tbl = xs_hbm.at[cid]

  def step(j, carry):
    pltpu.async_copy(tbl.at[src_v.at[j]], buf0, sg0).wait()
    pltpu.sync_copy(buf0, acc.at[dst_v.at[j]], add=True)
    return carry

  lax.fori_loop(0, CH, step, 0)
  plsc.subcore_barrier()
  pltpu.sync_copy(acc.at[sl], out_sum.at[cid, sl])


_segsum = pl.kernel(
    _segsum_body,
    out_type=jax.ShapeDtypeStruct((NC, ACC_N, D2), jnp.float32),
    mesh=_mesh,
    scratch_types=[
        pltpu.VMEM((CH, L), jnp.int32),
        pltpu.VMEM((CH, L), jnp.int32),
        pltpu.VMEM((L, D2), jnp.float32),
        pltpu.VMEM((L, D2), jnp.float32),
        pltpu.VMEM_SHARED((ACC_N, D2), jnp.float32),
        pltpu.SemaphoreType.DMA,
        pltpu.SemaphoreType.DMA,
    ],
    compiler_params=pltpu.CompilerParams(use_tc_tiling_on_sc=False),
)


def _count_body(dst_hbm, zc_hbm, ones_hbm, out_cnt, dst_v, ones_v, cacc):
  cid = lax.axis_index("c")
  sid = lax.axis_index("s")
  tid = cid * NS + sid

  pltpu.sync_copy(dst_hbm.at[tid], dst_v)
  pltpu.sync_copy(ones_hbm, ones_v)
  sl = pl.ds(sid * STRIPE, STRIPE)
  pltpu.sync_copy(zc_hbm.at[sl], cacc.at[sl])
  plsc.subcore_barrier()

  def step(j, carry):
    pltpu.sync_copy(ones_v, cacc.at[dst_v.at[j]], add=True)
    return carry

  lax.fori_loop(0, CHC, step, 0)
  plsc.subcore_barrier()
  pltpu.sync_copy(cacc.at[sl], out_cnt.at[cid, sl])


_count = pl.kernel(
    _count_body,
    out_type=jax.ShapeDtypeStruct((NC, ACC_N, CW), jnp.float32),
    mesh=_mesh,
    scratch_types=[
        pltpu.VMEM((CHC, L), jnp.int32),
        pltpu.VMEM((L, CW), jnp.float32),
        pltpu.VMEM_SHARED((ACC_N, CW), jnp.float32),
    ],
    compiler_params=pltpu.CompilerParams(use_tc_tiling_on_sc=False),
)

R = 400


def _combine_body(p_ref, c_ref, x_ref, w_ref, b_ref, o_ref):
  s = jnp.concatenate([p_ref[0], p_ref[1]], axis=1)
  cnt = c_ref[0, :, 0:1] + c_ref[1, :, 0:1]
  agg = s / jnp.maximum(cnt, 1.0)
  cat = jnp.concatenate([agg, x_ref[...]], axis=1)
  o = lax.dot_general(cat, w_ref[...], (((1,), (0,)), ((), ())),
                      preferred_element_type=jnp.float32,
                      precision=lax.Precision.HIGHEST)
  o_ref[...] = jnp.maximum(o + b_ref[...], 0.0)


_combine = pl.pallas_call(
    _combine_body,
    grid=(N // R,),
    in_specs=[
        pl.BlockSpec((NC, R, D2), lambda i: (0, i, 0)),
        pl.BlockSpec((NC, R, CW), lambda i: (0, i, 0)),
        pl.BlockSpec((R, D), lambda i: (i, 0)),
        pl.BlockSpec((2 * D, D), lambda i: (0, 0)),
        pl.BlockSpec((1, D), lambda i: (0, 0)),
    ],
    out_specs=pl.BlockSpec((R, D), lambda i: (i, 0)),
    out_shape=jax.ShapeDtypeStruct((N, D), jnp.float32),
)


def _split(h):
  return jnp.stack([h[:, :D2], h[:, D2:]], axis=0)


def kernel(x, edge_index, edge_attr, W1l, W1r, b1, W2l, W2r, b2):
  npad = E_PAD - E
  src = jnp.concatenate(
      [edge_index[0].astype(jnp.int32),
       jnp.arange(npad, dtype=jnp.int32) % N])
  dst = jnp.concatenate(
      [edge_index[1].astype(jnp.int32),
       N + (jnp.arange(npad, dtype=jnp.int32) % (ACC_N - N))])
  src_s = src.reshape(NS, CH, L)
  dst_s = dst.reshape(NS, CH, L)
  dst_c = dst.reshape(NW, CHC, L)
  z = jnp.zeros((ACC_N, D2), jnp.float32)
  zc = jnp.zeros((ACC_N, CW), jnp.float32)
  ones = jnp.ones((L, CW), jnp.float32)
  w1 = jnp.concatenate([W1l.T, W1r.T], axis=0)
  w2 = jnp.concatenate([W2l.T, W2r.T], axis=0)

  cnt = _count(dst_c, zc, ones)
  p1 = _segsum(_split(x), src_s, dst_s, z)
  h1 = _combine(p1, cnt, x, w1, b1.reshape(1, D))
  p2 = _segsum(_split(h1), src_s, dst_s, z)
  h2 = _combine(p2, cnt, h1, w2, b2.reshape(1, D))
  return h2


# speedup vs baseline: 8.4558x; 1.3083x over previous
import jax
import jax.numpy as jnp
from jax import lax
from jax.experimental import pallas as pl
from jax.experimental.pallas import tpu as pltpu
from jax.experimental.pallas import tpu_sc as plsc

N = 10000
D = 128
D2 = D // 2
E = 320000

NC = 2
NS = 16
NW = NC * NS
L = 128
U = 4
E_PAD = 327680
CH = E_PAD // (NS * L)
CHC = E_PAD // (NW * L)
ACC_N = 10112
STRIPE = ACC_N // NS
CW = 16

_mesh = plsc.VectorSubcoreMesh(core_axis_name="c", subcore_axis_name="s")


def _segsum_body(xs_hbm, src_hbm, dst_hbm, z_hbm,
                 out_sum, src_v, dst_v, bufs, acc, gsems, ssems):
  cid = lax.axis_index("c")
  sid = lax.axis_index("s")

  pltpu.sync_copy(src_hbm.at[sid], src_v)
  pltpu.sync_copy(dst_hbm.at[sid], dst_v)
  sl = pl.ds(sid * STRIPE, STRIPE)
  pltpu.sync_copy(z_hbm.at[sl], acc.at[sl])
  plsc.subcore_barrier()

  tbl = xs_hbm.at[cid]

  def step(t, carry):
    base = U * t
    ds = [pltpu.async_copy(tbl.at[src_v.at[base + k]], bufs.at[k], gsems.at[k])
          for k in range(U)]
    for k in range(U):
      ds[k].wait()
      pltpu.sync_copy(bufs.at[k], acc.at[dst_v.at[base + k]], add=True)
    return carry

  lax.fori_loop(0, CH // U, step, 0)
  plsc.subcore_barrier()
  pltpu.sync_copy(acc.at[sl], out_sum.at[cid, sl])


_segsum = pl.kernel(
    _segsum_body,
    out_type=jax.ShapeDtypeStruct((NC, ACC_N, D2), jnp.float32),
    mesh=_mesh,
    scratch_types=[
        pltpu.VMEM((CH, L), jnp.int32),
        pltpu.VMEM((CH, L), jnp.int32),
        pltpu.VMEM((U, L, D2), jnp.float32),
        pltpu.VMEM_SHARED((ACC_N, D2), jnp.float32),
        pltpu.SemaphoreType.DMA((U,)),
        pltpu.SemaphoreType.DMA((U,)),
    ],
    compiler_params=pltpu.CompilerParams(use_tc_tiling_on_sc=False),
)


def _count_body(dst_hbm, zc_hbm, ones_hbm, out_cnt, dst_v, ones_v, cacc):
  cid = lax.axis_index("c")
  sid = lax.axis_index("s")
  tid = cid * NS + sid

  pltpu.sync_copy(dst_hbm.at[tid], dst_v)
  pltpu.sync_copy(ones_hbm, ones_v)
  sl = pl.ds(sid * STRIPE, STRIPE)
  pltpu.sync_copy(zc_hbm.at[sl], cacc.at[sl])
  plsc.subcore_barrier()

  def step(j, carry):
    pltpu.sync_copy(ones_v, cacc.at[dst_v.at[j]], add=True)
    return carry

  lax.fori_loop(0, CHC, step, 0)
  plsc.subcore_barrier()
  pltpu.sync_copy(cacc.at[sl], out_cnt.at[cid, sl])


_count = pl.kernel(
    _count_body,
    out_type=jax.ShapeDtypeStruct((NC, ACC_N, CW), jnp.float32),
    mesh=_mesh,
    scratch_types=[
        pltpu.VMEM((CHC, L), jnp.int32),
        pltpu.VMEM((L, CW), jnp.float32),
        pltpu.VMEM_SHARED((ACC_N, CW), jnp.float32),
    ],
    compiler_params=pltpu.CompilerParams(use_tc_tiling_on_sc=False),
)

R = 400


def _combine_body(p_ref, c_ref, x_ref, w_ref, b_ref, o_ref):
  s = jnp.concatenate([p_ref[0], p_ref[1]], axis=1)
  cnt = c_ref[0, :, 0:1] + c_ref[1, :, 0:1]
  agg = s / jnp.maximum(cnt, 1.0)
  cat = jnp.concatenate([agg, x_ref[...]], axis=1)
  o = lax.dot_general(cat, w_ref[...], (((1,), (0,)), ((), ())),
                      preferred_element_type=jnp.float32,
                      precision=lax.Precision.HIGHEST)
  o_ref[...] = jnp.maximum(o + b_ref[...], 0.0)


_combine = pl.pallas_call(
    _combine_body,
    grid=(N // R,),
    in_specs=[
        pl.BlockSpec((NC, R, D2), lambda i: (0, i, 0)),
        pl.BlockSpec((NC, R, CW), lambda i: (0, i, 0)),
        pl.BlockSpec((R, D), lambda i: (i, 0)),
        pl.BlockSpec((2 * D, D), lambda i: (0, 0)),
        pl.BlockSpec((1, D), lambda i: (0, 0)),
    ],
    out_specs=pl.BlockSpec((R, D), lambda i: (i, 0)),
    out_shape=jax.ShapeDtypeStruct((N, D), jnp.float32),
)


def _split(h):
  return jnp.stack([h[:, :D2], h[:, D2:]], axis=0)


def kernel(x, edge_index, edge_attr, W1l, W1r, b1, W2l, W2r, b2):
  npad = E_PAD - E
  src = jnp.concatenate(
      [edge_index[0].astype(jnp.int32),
       jnp.arange(npad, dtype=jnp.int32) % N])
  dst = jnp.concatenate(
      [edge_index[1].astype(jnp.int32),
       N + (jnp.arange(npad, dtype=jnp.int32) % (ACC_N - N))])
  src_s = src.reshape(NS, CH, L)
  dst_s = dst.reshape(NS, CH, L)
  dst_c = dst.reshape(NW, CHC, L)
  z = jnp.zeros((ACC_N, D2), jnp.float32)
  zc = jnp.zeros((ACC_N, CW), jnp.float32)
  ones = jnp.ones((L, CW), jnp.float32)
  w1 = jnp.concatenate([W1l.T, W1r.T], axis=0)
  w2 = jnp.concatenate([W2l.T, W2r.T], axis=0)

  cnt = _count(dst_c, zc, ones)
  p1 = _segsum(_split(x), src_s, dst_s, z)
  h1 = _combine(p1, cnt, x, w1, b1.reshape(1, D))
  p2 = _segsum(_split(h1), src_s, dst_s, z)
  h2 = _combine(p2, cnt, h1, w2, b2.reshape(1, D))
  return h2


# speedup vs baseline: 8.7150x; 1.0307x over previous
import jax
import jax.numpy as jnp
from jax import lax
from jax.experimental import pallas as pl
from jax.experimental.pallas import tpu as pltpu
from jax.experimental.pallas import tpu_sc as plsc

N = 10000
D = 128
D2 = D // 2
E = 320000

NC = 2
NS = 16
NW = NC * NS
L = 128
LC = 128
U = 5
E_PAD = 327680
CH = E_PAD // (NS * L)
CHC = E_PAD // (NW * LC)
ACC_N = 10112
STRIPE = ACC_N // NS
CW = 16

_mesh = plsc.VectorSubcoreMesh(core_axis_name="c", subcore_axis_name="s")


def _segsum_body(xs_hbm, src_hbm, dst_hbm, z_hbm,
                 out_sum, src_v, dst_v, bufs, acc, gsems, ssems):
  cid = lax.axis_index("c")
  sid = lax.axis_index("s")

  pltpu.sync_copy(src_hbm.at[sid], src_v)
  pltpu.sync_copy(dst_hbm.at[sid], dst_v)
  sl = pl.ds(sid * STRIPE, STRIPE)
  pltpu.sync_copy(z_hbm.at[sl], acc.at[sl])
  plsc.subcore_barrier()

  tbl = xs_hbm.at[cid]

  def step(t, carry):
    base = U * t
    ds = [pltpu.async_copy(tbl.at[src_v.at[base + k]], bufs.at[k], gsems.at[k])
          for k in range(U)]
    for k in range(U):
      ds[k].wait()
      pltpu.sync_copy(bufs.at[k], acc.at[dst_v.at[base + k]], add=True)
    return carry

  lax.fori_loop(0, CH // U, step, 0)
  plsc.subcore_barrier()
  pltpu.sync_copy(acc.at[sl], out_sum.at[cid, sl])


_segsum = pl.kernel(
    _segsum_body,
    out_type=jax.ShapeDtypeStruct((NC, ACC_N, D2), jnp.float32),
    mesh=_mesh,
    scratch_types=[
        pltpu.VMEM((CH, L), jnp.int32),
        pltpu.VMEM((CH, L), jnp.int32),
        pltpu.VMEM((U, L, D2), jnp.float32),
        pltpu.VMEM_SHARED((ACC_N, D2), jnp.float32),
        pltpu.SemaphoreType.DMA((U,)),
        pltpu.SemaphoreType.DMA((U,)),
    ],
    compiler_params=pltpu.CompilerParams(use_tc_tiling_on_sc=False),
)


def _count_body(dst_hbm, zc_hbm, ones_hbm, out_cnt, dst_v, ones_v, cacc):
  cid = lax.axis_index("c")
  sid = lax.axis_index("s")
  tid = cid * NS + sid

  pltpu.sync_copy(dst_hbm.at[tid], dst_v)
  pltpu.sync_copy(ones_hbm, ones_v)
  sl = pl.ds(sid * STRIPE, STRIPE)
  pltpu.sync_copy(zc_hbm.at[sl], cacc.at[sl])
  plsc.subcore_barrier()

  def step(j, carry):
    pltpu.sync_copy(ones_v, cacc.at[dst_v.at[j]], add=True)
    return carry

  lax.fori_loop(0, CHC, step, 0)
  plsc.subcore_barrier()
  pltpu.sync_copy(cacc.at[sl], out_cnt.at[cid, sl])


_count = pl.kernel(
    _count_body,
    out_type=jax.ShapeDtypeStruct((NC, ACC_N, CW), jnp.float32),
    mesh=_mesh,
    scratch_types=[
        pltpu.VMEM((CHC, LC), jnp.int32),
        pltpu.VMEM((LC, CW), jnp.float32),
        pltpu.VMEM_SHARED((ACC_N, CW), jnp.float32),
    ],
    compiler_params=pltpu.CompilerParams(use_tc_tiling_on_sc=False),
)

R = 400


def _combine_body(p_ref, c_ref, x_ref, w_ref, b_ref, o_ref):
  s = jnp.concatenate([p_ref[0], p_ref[1]], axis=1)
  cnt = c_ref[0, :, 0:1] + c_ref[1, :, 0:1]
  agg = s / jnp.maximum(cnt, 1.0)
  cat = jnp.concatenate([agg, x_ref[...]], axis=1)
  o = lax.dot_general(cat, w_ref[...], (((1,), (0,)), ((), ())),
                      preferred_element_type=jnp.float32,
                      precision=lax.Precision.HIGHEST)
  o_ref[...] = jnp.maximum(o + b_ref[...], 0.0)


_combine = pl.pallas_call(
    _combine_body,
    grid=(N // R,),
    in_specs=[
        pl.BlockSpec((NC, R, D2), lambda i: (0, i, 0)),
        pl.BlockSpec((NC, R, CW), lambda i: (0, i, 0)),
        pl.BlockSpec((R, D), lambda i: (i, 0)),
        pl.BlockSpec((2 * D, D), lambda i: (0, 0)),
        pl.BlockSpec((1, D), lambda i: (0, 0)),
    ],
    out_specs=pl.BlockSpec((R, D), lambda i: (i, 0)),
    out_shape=jax.ShapeDtypeStruct((N, D), jnp.float32),
)


def _split(h):
  return jnp.stack([h[:, :D2], h[:, D2:]], axis=0)


def kernel(x, edge_index, edge_attr, W1l, W1r, b1, W2l, W2r, b2):
  npad = E_PAD - E
  src = jnp.concatenate(
      [edge_index[0].astype(jnp.int32),
       jnp.arange(npad, dtype=jnp.int32) % N])
  dst = jnp.concatenate(
      [edge_index[1].astype(jnp.int32),
       N + (jnp.arange(npad, dtype=jnp.int32) % (ACC_N - N))])
  src_s = src.reshape(NS, CH, L)
  dst_s = dst.reshape(NS, CH, L)
  dst_c = dst.reshape(NW, CHC, LC)
  z = jnp.zeros((ACC_N, D2), jnp.float32)
  zc = jnp.zeros((ACC_N, CW), jnp.float32)
  ones = jnp.ones((LC, CW), jnp.float32)
  w1 = jnp.concatenate([W1l.T, W1r.T], axis=0)
  w2 = jnp.concatenate([W2l.T, W2r.T], axis=0)

  cnt = _count(dst_c, zc, ones)
  p1 = _segsum(_split(x), src_s, dst_s, z)
  h1 = _combine(p1, cnt, x, w1, b1.reshape(1, D))
  p2 = _segsum(_split(h1), src_s, dst_s, z)
  h2 = _combine(p2, cnt, h1, w2, b2.reshape(1, D))
  return h2


# speedup vs baseline: 8.7900x; 1.0086x over previous
import jax
import jax.numpy as jnp
from jax import lax
from jax.experimental import pallas as pl
from jax.experimental.pallas import tpu as pltpu
from jax.experimental.pallas import tpu_sc as plsc

N = 10000
D = 128
D2 = D // 2
E = 320000

NC = 2
NS = 16
NW = NC * NS
L = 128
LC = 128
U = 5
E_PAD = 327680
CH = E_PAD // (NS * L)
CHC = E_PAD // (NW * LC)
ACC_N = 10112
STRIPE = ACC_N // NS
CW = 16

_mesh = plsc.VectorSubcoreMesh(core_axis_name="c", subcore_axis_name="s")


def _segsum_body(xs_hbm, src_hbm, dst_hbm, z_hbm,
                 out_sum, src_v, dst_v, bufs, acc, gsems, ssems):
  cid = lax.axis_index("c")
  sid = lax.axis_index("s")

  pltpu.sync_copy(src_hbm.at[sid], src_v)
  pltpu.sync_copy(dst_hbm.at[sid], dst_v)
  sl = pl.ds(sid * STRIPE, STRIPE)
  pltpu.sync_copy(z_hbm.at[sl], acc.at[sl])
  plsc.subcore_barrier()

  tbl = xs_hbm.at[cid]

  def step(t, carry):
    base = U * t
    ds = [pltpu.async_copy(tbl.at[src_v.at[base + k]], bufs.at[k], gsems.at[k])
          for k in range(U)]
    for k in range(U):
      ds[k].wait()
      pltpu.sync_copy(bufs.at[k], acc.at[dst_v.at[base + k]], add=True)
    return carry

  lax.fori_loop(0, CH // U, step, 0)
  plsc.subcore_barrier()
  pltpu.sync_copy(acc.at[sl], out_sum.at[cid, sl])


_segsum = pl.kernel(
    _segsum_body,
    out_type=jax.ShapeDtypeStruct((NC, ACC_N, D2), jnp.float32),
    mesh=_mesh,
    scratch_types=[
        pltpu.VMEM((CH, L), jnp.int32),
        pltpu.VMEM((CH, L), jnp.int32),
        pltpu.VMEM((U, L, D2), jnp.float32),
        pltpu.VMEM_SHARED((ACC_N, D2), jnp.float32),
        pltpu.SemaphoreType.DMA((U,)),
        pltpu.SemaphoreType.DMA((U,)),
    ],
    compiler_params=pltpu.CompilerParams(use_tc_tiling_on_sc=False),
)


def _count_body(dst_hbm, zc_hbm, ones_hbm, out_cnt, dst_v, ones_v, cacc):
  cid = lax.axis_index("c")
  sid = lax.axis_index("s")
  tid = cid * NS + sid

  pltpu.sync_copy(dst_hbm.at[tid], dst_v)
  pltpu.sync_copy(ones_hbm, ones_v)
  sl = pl.ds(sid * STRIPE, STRIPE)
  pltpu.sync_copy(zc_hbm.at[sl], cacc.at[sl])
  plsc.subcore_barrier()

  def step(j, carry):
    pltpu.sync_copy(ones_v, cacc.at[dst_v.at[j]], add=True)
    return carry

  lax.fori_loop(0, CHC, step, 0)
  plsc.subcore_barrier()
  pltpu.sync_copy(cacc.at[sl], out_cnt.at[cid, sl])


_count = pl.kernel(
    _count_body,
    out_type=jax.ShapeDtypeStruct((NC, ACC_N, CW), jnp.float32),
    mesh=_mesh,
    scratch_types=[
        pltpu.VMEM((CHC, LC), jnp.int32),
        pltpu.VMEM((LC, CW), jnp.float32),
        pltpu.VMEM_SHARED((ACC_N, CW), jnp.float32),
    ],
    compiler_params=pltpu.CompilerParams(use_tc_tiling_on_sc=False),
)

R = 400


def _make_combine(split_out):
  def body(p_ref, c_ref, x_ref, w_ref, b_ref, o_ref):
    s = jnp.concatenate([p_ref[0], p_ref[1]], axis=1)
    cnt = c_ref[0, :, 0:1] + c_ref[1, :, 0:1]
    agg = s / jnp.maximum(cnt, 1.0)
    xcat = jnp.concatenate([x_ref[0], x_ref[1]], axis=1)
    cat = jnp.concatenate([agg, xcat], axis=1)
    o = lax.dot_general(cat, w_ref[...], (((1,), (0,)), ((), ())),
                        preferred_element_type=jnp.float32,
                        precision=lax.Precision.HIGHEST)
    o = jnp.maximum(o + b_ref[...], 0.0)
    if split_out:
      o_ref[0] = o[:, :D2]
      o_ref[1] = o[:, D2:]
    else:
      o_ref[...] = o

  if split_out:
    out_spec = pl.BlockSpec((NC, R, D2), lambda i: (0, i, 0))
    out_shape = jax.ShapeDtypeStruct((NC, N, D2), jnp.float32)
  else:
    out_spec = pl.BlockSpec((R, D), lambda i: (i, 0))
    out_shape = jax.ShapeDtypeStruct((N, D), jnp.float32)

  return pl.pallas_call(
      body,
      grid=(N // R,),
      in_specs=[
          pl.BlockSpec((NC, R, D2), lambda i: (0, i, 0)),
          pl.BlockSpec((NC, R, CW), lambda i: (0, i, 0)),
          pl.BlockSpec((NC, R, D2), lambda i: (0, i, 0)),
          pl.BlockSpec((2 * D, D), lambda i: (0, 0)),
          pl.BlockSpec((1, D), lambda i: (0, 0)),
      ],
      out_specs=out_spec,
      out_shape=out_shape,
  )


_combine_split = _make_combine(True)
_combine_full = _make_combine(False)


def _split(h):
  return jnp.stack([h[:, :D2], h[:, D2:]], axis=0)


def kernel(x, edge_index, edge_attr, W1l, W1r, b1, W2l, W2r, b2):
  npad = E_PAD - E
  src = jnp.concatenate(
      [edge_index[0].astype(jnp.int32),
       jnp.arange(npad, dtype=jnp.int32) % N])
  dst = jnp.concatenate(
      [edge_index[1].astype(jnp.int32),
       N + (jnp.arange(npad, dtype=jnp.int32) % (ACC_N - N))])
  src_s = src.reshape(NS, CH, L)
  dst_s = dst.reshape(NS, CH, L)
  dst_c = dst.reshape(NW, CHC, LC)
  z = jnp.zeros((ACC_N, D2), jnp.float32)
  zc = jnp.zeros((ACC_N, CW), jnp.float32)
  ones = jnp.ones((LC, CW), jnp.float32)
  w1 = jnp.concatenate([W1l.T, W1r.T], axis=0)
  w2 = jnp.concatenate([W2l.T, W2r.T], axis=0)

  xs = _split(x)
  cnt = _count(dst_c, zc, ones)
  p1 = _segsum(xs, src_s, dst_s, z)
  hs1 = _combine_split(p1, cnt, xs, w1, b1.reshape(1, D))
  p2 = _segsum(hs1, src_s, dst_s, z)
  h2 = _combine_full(p2, cnt, hs1, w2, b2.reshape(1, D))
  return h2


# speedup vs baseline: 9.2707x; 1.0547x over previous
import jax
import jax.numpy as jnp
from jax import lax
from jax.experimental import pallas as pl
from jax.experimental.pallas import tpu as pltpu
from jax.experimental.pallas import tpu_sc as plsc

N = 10000
D = 128
D2 = D // 2
E = 320000

NC = 2
NS = 16
NW = NC * NS
L = 128
LC = 128
U = 5
E_PAD = 327680
CH = E_PAD // (NS * L)
CHC = E_PAD // (NW * LC)
ACC_N = 10112
STRIPE = ACC_N // NS
CW = 16

_mesh = plsc.VectorSubcoreMesh(core_axis_name="c", subcore_axis_name="s")


def _segsum_body(xs_hbm, src_hbm, dst_hbm, z_hbm,
                 out_sum, src_v, dst_v, bufs, acc, gsems, ssems):
  cid = lax.axis_index("c")
  sid = lax.axis_index("s")

  pltpu.sync_copy(src_hbm.at[sid], src_v)
  pltpu.sync_copy(dst_hbm.at[sid], dst_v)
  sl = pl.ds(sid * STRIPE, STRIPE)
  pltpu.sync_copy(z_hbm.at[sl], acc.at[sl])
  plsc.subcore_barrier()

  tbl = xs_hbm.at[cid]

  def step(t, carry):
    base = U * t
    ds = [pltpu.async_copy(tbl.at[src_v.at[base + k]], bufs.at[k], gsems.at[k])
          for k in range(U)]
    for k in range(U):
      ds[k].wait()
      pltpu.sync_copy(bufs.at[k], acc.at[dst_v.at[base + k]], add=True)
    return carry

  lax.fori_loop(0, CH // U, step, 0)
  plsc.subcore_barrier()
  pltpu.sync_copy(acc.at[sl], out_sum.at[cid, sl])


_segsum = pl.kernel(
    _segsum_body,
    out_type=jax.ShapeDtypeStruct((NC, ACC_N, D2), jnp.float32),
    mesh=_mesh,
    scratch_types=[
        pltpu.VMEM((CH, L), jnp.int32),
        pltpu.VMEM((CH, L), jnp.int32),
        pltpu.VMEM((U, L, D2), jnp.float32),
        pltpu.VMEM_SHARED((ACC_N, D2), jnp.float32),
        pltpu.SemaphoreType.DMA((U,)),
        pltpu.SemaphoreType.DMA((U,)),
    ],
    compiler_params=pltpu.CompilerParams(use_tc_tiling_on_sc=False),
)


def _count_body(dst_hbm, zc_hbm, ones_hbm, out_cnt, dst_v, ones_v, cacc):
  cid = lax.axis_index("c")
  sid = lax.axis_index("s")
  tid = cid * NS + sid

  pltpu.sync_copy(dst_hbm.at[tid], dst_v)
  pltpu.sync_copy(ones_hbm, ones_v)
  sl = pl.ds(sid * STRIPE, STRIPE)
  pltpu.sync_copy(zc_hbm.at[sl], cacc.at[sl])
  plsc.subcore_barrier()

  def step(j, carry):
    pltpu.sync_copy(ones_v, cacc.at[dst_v.at[j]], add=True)
    return carry

  lax.fori_loop(0, CHC, step, 0)
  plsc.subcore_barrier()
  pltpu.sync_copy(cacc.at[sl], out_cnt.at[cid, sl])


_count = pl.kernel(
    _count_body,
    out_type=jax.ShapeDtypeStruct((NC, ACC_N, CW), jnp.float32),
    mesh=_mesh,
    scratch_types=[
        pltpu.VMEM((CHC, LC), jnp.int32),
        pltpu.VMEM((LC, CW), jnp.float32),
        pltpu.VMEM_SHARED((ACC_N, CW), jnp.float32),
    ],
    compiler_params=pltpu.CompilerParams(use_tc_tiling_on_sc=False),
)

R = 2000


def _make_combine(split_out):
  def body(p_ref, c_ref, x_ref, w_ref, b_ref, o_ref):
    s = jnp.concatenate([p_ref[0], p_ref[1]], axis=1)
    cnt = c_ref[0, :, 0:1] + c_ref[1, :, 0:1]
    agg = s / jnp.maximum(cnt, 1.0)
    xcat = jnp.concatenate([x_ref[0], x_ref[1]], axis=1)
    cat = jnp.concatenate([agg, xcat], axis=1)
    o = lax.dot_general(cat, w_ref[...], (((1,), (0,)), ((), ())),
                        preferred_element_type=jnp.float32,
                        precision=lax.Precision.HIGHEST)
    o = jnp.maximum(o + b_ref[...], 0.0)
    if split_out:
      o_ref[0] = o[:, :D2]
      o_ref[1] = o[:, D2:]
    else:
      o_ref[...] = o

  if split_out:
    out_spec = pl.BlockSpec((NC, R, D2), lambda i: (0, i, 0))
    out_shape = jax.ShapeDtypeStruct((NC, N, D2), jnp.float32)
  else:
    out_spec = pl.BlockSpec((R, D), lambda i: (i, 0))
    out_shape = jax.ShapeDtypeStruct((N, D), jnp.float32)

  return pl.pallas_call(
      body,
      grid=(N // R,),
      in_specs=[
          pl.BlockSpec((NC, R, D2), lambda i: (0, i, 0)),
          pl.BlockSpec((NC, R, CW), lambda i: (0, i, 0)),
          pl.BlockSpec((NC, R, D2), lambda i: (0, i, 0)),
          pl.BlockSpec((2 * D, D), lambda i: (0, 0)),
          pl.BlockSpec((1, D), lambda i: (0, 0)),
      ],
      out_specs=out_spec,
      out_shape=out_shape,
  )


_combine_split = _make_combine(True)
_combine_full = _make_combine(False)


def _split(h):
  return jnp.stack([h[:, :D2], h[:, D2:]], axis=0)


def kernel(x, edge_index, edge_attr, W1l, W1r, b1, W2l, W2r, b2):
  npad = E_PAD - E
  src = jnp.concatenate(
      [edge_index[0].astype(jnp.int32),
       jnp.arange(npad, dtype=jnp.int32) % N])
  dst = jnp.concatenate(
      [edge_index[1].astype(jnp.int32),
       N + (jnp.arange(npad, dtype=jnp.int32) % (ACC_N - N))])
  src_s = src.reshape(NS, CH, L)
  dst_s = dst.reshape(NS, CH, L)
  dst_c = dst.reshape(NW, CHC, LC)
  z = jnp.zeros((ACC_N, D2), jnp.float32)
  zc = jnp.zeros((ACC_N, CW), jnp.float32)
  ones = jnp.ones((LC, CW), jnp.float32)
  w1 = jnp.concatenate([W1l.T, W1r.T], axis=0)
  w2 = jnp.concatenate([W2l.T, W2r.T], axis=0)

  xs = _split(x)
  cnt = _count(dst_c, zc, ones)
  p1 = _segsum(xs, src_s, dst_s, z)
  hs1 = _combine_split(p1, cnt, xs, w1, b1.reshape(1, D))
  p2 = _segsum(hs1, src_s, dst_s, z)
  h2 = _combine_full(p2, cnt, hs1, w2, b2.reshape(1, D))
  return h2
